# baseline (speedup 1.0000x reference)
import jax
import jax.numpy as jnp
from jax import lax
from jax.experimental import pallas as pl
from jax.experimental.pallas import tpu as pltpu
from jax.experimental.pallas import tpu_sc as plsc

BATCH = 16384
LATENT = 64
INV_ROWS = 1000000
STK_ROWS = 100000
NC = 2
NS = 16
NW = NC * NS

INV_R = 31232
INV_CW = 512
INV_NCH = INV_R // INV_CW
INV_TAIL = (INV_ROWS // 128) * 128
INV_G = 3

STK_R = 3072
STK_CW = 128
STK_NCH = STK_R // STK_CW
STK_TAIL = (STK_ROWS // 128) * 128
STK_G = 4

TAIL_G = 2
LOC_CAP = 1072
CH_CAP = 80
NIDX = BATCH // 16
DUMP = BATCH


def _count(mask):
    return jnp.sum(jnp.where(mask, 1, 0))


def _gather_body(inv_idx, pos_idx, neg_idx, inv_tab, stk_tab,
                 tail_inv, tail_stk,
                 g_inv, g_pos, g_neg,
                 idx_v, locr_a, locb_a, locr_b, locb_b,
                 chunk, chr_, chb_, stage, tail_buf):
    wid = lax.axis_index("s") * NC + lax.axis_index("c")
    last = wid == NW - 1
    iota = lax.broadcasted_iota(jnp.int32, (16,), 0)

    def select(idx_hbm, lo, hi, locr, locb):
        pltpu.sync_copy(idx_hbm, idx_v)

        def step(i, off):
            r = idx_v[pl.ds(i * 16, 16)]
            b = i * 16 + iota
            m = (r >= lo) & (r < hi)
            plsc.store_compressed(locr.at[pl.ds(off, 16)], r, mask=m)
            plsc.store_compressed(locb.at[pl.ds(off, 16)], b, mask=m)
            return off + _count(m)

        return lax.fori_loop(0, NIDX, step, 0)

    def consume(base, width, lists, max_g):
        for (locr, locb, cnt, out) in lists:
            def rebuild(v, moff):
                valid = (v * 16 + iota) < cnt
                r = locr[pl.ds(v * 16, 16)]
                b = locb[pl.ds(v * 16, 16)]
                m = valid & (r >= base) & (r < base + width)
                plsc.store_compressed(chr_.at[pl.ds(moff, 16)], r - base, mask=m)
                plsc.store_compressed(chb_.at[pl.ds(moff, 16)], b, mask=m)
                return moff + _count(m)

            k = lax.fori_loop(0, (cnt + 15) // 16, rebuild, 0)
            for g in range(max_g):
                @pl.when(g * 16 < k)
                def _do():
                    valid = (g * 16 + iota) < k
                    roff = jnp.where(valid, chr_[pl.ds(g * 16, 16)], 0)
                    bv = jnp.where(valid, chb_[pl.ds(g * 16, 16)], DUMP)
                    for d in range(LATENT):
                        dcol = jnp.full((16,), d, jnp.int32)
                        x = plsc.load_gather(chunk, [dcol, roff])
                        plsc.store_scatter(stage, [iota, dcol], x)
                    pltpu.sync_copy(stage, out.at[bv])

    def consume_tail(base, width, lists):
        for (locr, locb, cnt, out) in lists:
            def rebuild(v, moff):
                valid = (v * 16 + iota) < cnt
                r = locr[pl.ds(v * 16, 16)]
                b = locb[pl.ds(v * 16, 16)]
                m = valid & (r >= base) & (r < base + width)
                plsc.store_compressed(chr_.at[pl.ds(moff, 16)], r - base, mask=m)
                plsc.store_compressed(chb_.at[pl.ds(moff, 16)], b, mask=m)
                return moff + _count(m)

            k = lax.fori_loop(0, (cnt + 15) // 16, rebuild, 0)
            for g in range(TAIL_G):
                @pl.when(g * 16 < k)
                def _do():
                    valid = (g * 16 + iota) < k
                    roff = jnp.where(valid, chr_[pl.ds(g * 16, 16)], 0)
                    bv = jnp.where(valid, chb_[pl.ds(g * 16, 16)], DUMP)
                    for d in range(LATENT):
                        dcol = jnp.full((16,), d, jnp.int32)
                        x = plsc.load_gather(tail_buf, [roff, dcol])
                        plsc.store_scatter(stage, [iota, dcol], x)
                    pltpu.sync_copy(stage, out.at[bv])

    lo_i = wid * INV_R
    hi_i = jnp.where(last, INV_ROWS, lo_i + INV_R)
    cnt_i = select(inv_idx, lo_i, hi_i, locr_a, locb_a)
    ilists = [(locr_a, locb_a, cnt_i, g_inv)]

    def inv_chunk(c, carry):
        base = lo_i + c * INV_CW
        pltpu.sync_copy(inv_tab.at[:, pl.ds(base, INV_CW)],
                        chunk.at[:, pl.ds(0, INV_CW)])
        consume(base, INV_CW, ilists, INV_G)
        return carry

    lax.fori_loop(0, jnp.where(last, INV_NCH + 1, INV_NCH), inv_chunk, 0)

    @pl.when(last)
    def _inv_tail():
        pltpu.sync_copy(tail_inv, tail_buf.at[pl.ds(0, 64)])
        consume_tail(INV_TAIL, INV_ROWS - INV_TAIL, ilists)

    lo_s = wid * STK_R
    hi_s = jnp.where(last, STK_ROWS, lo_s + STK_R)
    cnt_p = select(pos_idx, lo_s, hi_s, locr_a, locb_a)
    cnt_n = select(neg_idx, lo_s, hi_s, locr_b, locb_b)
    slists = [(locr_a, locb_a, cnt_p, g_pos), (locr_b, locb_b, cnt_n, g_neg)]

    def stk_chunk(c, carry):
        base = lo_s + c * STK_CW
        pltpu.sync_copy(stk_tab.at[:, pl.ds(base, STK_CW)],
                        chunk.at[:, pl.ds(0, STK_CW)])
        consume(base, STK_CW, slists, STK_G)
        return carry

    lax.fori_loop(0, jnp.where(last, 37, STK_NCH), stk_chunk, 0)

    @pl.when(last)
    def _stk_tail():
        pltpu.sync_copy(tail_stk, tail_buf.at[pl.ds(0, 32)])
        consume_tail(STK_TAIL, STK_ROWS - STK_TAIL, slists)


_gather = pl.kernel(
    _gather_body,
    out_type=[
        jax.ShapeDtypeStruct((BATCH + 1, 128), jnp.float32),
        jax.ShapeDtypeStruct((BATCH + 1, 128), jnp.float32),
        jax.ShapeDtypeStruct((BATCH + 1, 128), jnp.float32),
    ],
    mesh=plsc.VectorSubcoreMesh(core_axis_name="c", subcore_axis_name="s"),
    compiler_params=pltpu.CompilerParams(
        needs_layout_passes=False, use_tc_tiling_on_sc=True
    ),
    scratch_types=[
        pltpu.VMEM((BATCH,), jnp.int32),
        pltpu.VMEM((LOC_CAP,), jnp.int32),
        pltpu.VMEM((LOC_CAP,), jnp.int32),
        pltpu.VMEM((LOC_CAP,), jnp.int32),
        pltpu.VMEM((LOC_CAP,), jnp.int32),
        pltpu.VMEM((LATENT, INV_CW), jnp.float32),
        pltpu.VMEM((CH_CAP,), jnp.int32),
        pltpu.VMEM((CH_CAP,), jnp.int32),
        pltpu.VMEM((16, 128), jnp.float32),
        pltpu.VMEM((64, LATENT), jnp.float32),
    ],
)

BR = 4096


def _dot_body(a_ref, p_ref, n_ref, op_ref, on_ref):
    a = a_ref[...]
    cols = lax.broadcasted_iota(jnp.int32, (BR, 128), 1)
    keep = cols < LATENT
    p = jnp.where(keep, a * p_ref[...], 0.0)
    n = jnp.where(keep, a * n_ref[...], 0.0)
    op_ref[...] = jnp.sum(p, axis=1).reshape(8, 512)
    on_ref[...] = jnp.sum(n, axis=1).reshape(8, 512)


_dot = pl.pallas_call(
    _dot_body,
    grid=(BATCH // BR,),
    in_specs=[
        pl.BlockSpec((BR, 128), lambda i: (i, 0)),
        pl.BlockSpec((BR, 128), lambda i: (i, 0)),
        pl.BlockSpec((BR, 128), lambda i: (i, 0)),
    ],
    out_specs=[
        pl.BlockSpec((8, 512), lambda i: (i, 0)),
        pl.BlockSpec((8, 512), lambda i: (i, 0)),
    ],
    out_shape=[
        jax.ShapeDtypeStruct((BATCH // 512, 512), jnp.float32),
        jax.ShapeDtypeStruct((BATCH // 512, 512), jnp.float32),
    ],
)


@jax.jit
def kernel(investor, stock_positive, stock_negative, embed_investor, embed_stock):
    inv_idx = investor.astype(jnp.int32)
    pos_idx = stock_positive.astype(jnp.int32)
    neg_idx = stock_negative.astype(jnp.int32)
    g_inv, g_pos, g_neg = _gather(
        inv_idx, pos_idx, neg_idx, embed_investor.T, embed_stock.T,
        embed_investor[INV_TAIL:], embed_stock[STK_TAIL:]
    )
    out_p, out_n = _dot(g_inv, g_pos, g_neg)
    return (out_p.reshape(BATCH), out_n.reshape(BATCH))


# speedup vs baseline: 1.0154x; 1.0154x over previous
import jax
import jax.numpy as jnp
from jax import lax
from jax.experimental import pallas as pl
from jax.experimental.pallas import tpu as pltpu
from jax.experimental.pallas import tpu_sc as plsc

BATCH = 16384
LATENT = 64
INV_ROWS = 1000000
STK_ROWS = 100000
NC = 2
NS = 16
NW = NC * NS

INV_R = 31232
INV_CW = 512
INV_NCH = INV_R // INV_CW
INV_TAIL = (INV_ROWS // 128) * 128
INV_G = 3

STK_R = 3072
STK_CW = 128
STK_NCH = STK_R // STK_CW
STK_TAIL = (STK_ROWS // 128) * 128
STK_G = 4

TAIL_G = 2
LOC_CAP = 1072
CH_CAP = 80
NIDX = BATCH // 16
DUMP = BATCH


def _count(mask):
    return jnp.sum(jnp.where(mask, 1, 0))


def _gather_body(inv_idx, pos_idx, neg_idx, inv_tab, stk_tab,
                 tail_inv, tail_stk,
                 g_inv, g_pos, g_neg,
                 idx_v, locr_a, locb_a, locr_b, locb_b,
                 chunk, chr_, chb_, stage, tail_buf, sem_a, sem_b):
    wid = lax.axis_index("s") * NC + lax.axis_index("c")
    last = wid == NW - 1
    iota = lax.broadcasted_iota(jnp.int32, (16,), 0)

    def select(idx_hbm, lo, hi, locr, locb):
        pltpu.sync_copy(idx_hbm, idx_v)

        def step(i, off):
            r = idx_v[pl.ds(i * 16, 16)]
            b = i * 16 + iota
            m = (r >= lo) & (r < hi)
            plsc.store_compressed(locr.at[pl.ds(off, 16)], r, mask=m)
            plsc.store_compressed(locb.at[pl.ds(off, 16)], b, mask=m)
            return off + _count(m)

        return lax.fori_loop(0, NIDX, step, 0)

    def consume(buf, base, width, lists, max_g):
        for (locr, locb, cnt, out) in lists:
            def rebuild(v, moff):
                valid = (v * 16 + iota) < cnt
                r = locr[pl.ds(v * 16, 16)]
                b = locb[pl.ds(v * 16, 16)]
                m = valid & (r >= base) & (r < base + width)
                plsc.store_compressed(chr_.at[pl.ds(moff, 16)], r - base, mask=m)
                plsc.store_compressed(chb_.at[pl.ds(moff, 16)], b, mask=m)
                return moff + _count(m)

            k = lax.fori_loop(0, (cnt + 15) // 16, rebuild, 0)
            for g in range(max_g):
                @pl.when(g * 16 < k)
                def _do():
                    valid = (g * 16 + iota) < k
                    roff = jnp.where(valid, chr_[pl.ds(g * 16, 16)], 0)
                    bv = jnp.where(valid, chb_[pl.ds(g * 16, 16)], DUMP)
                    for d in range(LATENT):
                        dcol = jnp.full((16,), d, jnp.int32)
                        x = plsc.load_gather(buf, [dcol, roff])
                        plsc.store_scatter(stage, [iota, dcol], x)
                    pltpu.sync_copy(stage, out.at[bv])

    def consume_tail(base, width, lists):
        for (locr, locb, cnt, out) in lists:
            def rebuild(v, moff):
                valid = (v * 16 + iota) < cnt
                r = locr[pl.ds(v * 16, 16)]
                b = locb[pl.ds(v * 16, 16)]
                m = valid & (r >= base) & (r < base + width)
                plsc.store_compressed(chr_.at[pl.ds(moff, 16)], r - base, mask=m)
                plsc.store_compressed(chb_.at[pl.ds(moff, 16)], b, mask=m)
                return moff + _count(m)

            k = lax.fori_loop(0, (cnt + 15) // 16, rebuild, 0)
            for g in range(TAIL_G):
                @pl.when(g * 16 < k)
                def _do():
                    valid = (g * 16 + iota) < k
                    roff = jnp.where(valid, chr_[pl.ds(g * 16, 16)], 0)
                    bv = jnp.where(valid, chb_[pl.ds(g * 16, 16)], DUMP)
                    for d in range(LATENT):
                        dcol = jnp.full((16,), d, jnp.int32)
                        x = plsc.load_gather(tail_buf, [roff, dcol])
                        plsc.store_scatter(stage, [iota, dcol], x)
                    pltpu.sync_copy(stage, out.at[bv])

    def stream(tab, lo, cw, nch, lists, max_g):
        def fire(c, slot_sem):
            slot, sem = slot_sem
            pltpu.async_copy(tab.at[:, pl.ds(lo + c * cw, cw)],
                             chunk.at[slot, :, pl.ds(0, cw)], sem)

        def wait(c, slot_sem):
            slot, sem = slot_sem
            pltpu.make_async_copy(tab.at[:, pl.ds(lo + c * cw, cw)],
                                  chunk.at[slot, :, pl.ds(0, cw)], sem).wait()

        fire(0, (0, sem_a))

        def body(c, carry):
            even = lax.rem(c, 2) == 0

            @pl.when((c + 1 < nch) & even)
            def _f1():
                fire(c + 1, (1, sem_b))

            @pl.when((c + 1 < nch) & jnp.logical_not(even))
            def _f0():
                fire(c + 1, (0, sem_a))

            @pl.when(even)
            def _w0():
                wait(c, (0, sem_a))

            @pl.when(jnp.logical_not(even))
            def _w1():
                wait(c, (1, sem_b))

            slot = lax.rem(c, 2)
            consume(chunk.at[slot], lo + c * cw, cw, lists, max_g)
            return carry

        lax.fori_loop(0, nch, body, 0)

    lo_i = wid * INV_R
    hi_i = jnp.where(last, INV_ROWS, lo_i + INV_R)
    cnt_i = select(inv_idx, lo_i, hi_i, locr_a, locb_a)
    ilists = [(locr_a, locb_a, cnt_i, g_inv)]

    stream(inv_tab, lo_i, INV_CW,
           jnp.where(last, INV_NCH + 1, INV_NCH), ilists, INV_G)

    @pl.when(last)
    def _inv_tail():
        pltpu.sync_copy(tail_inv, tail_buf.at[pl.ds(0, 64)])
        consume_tail(INV_TAIL, INV_ROWS - INV_TAIL, ilists)

    lo_s = wid * STK_R
    hi_s = jnp.where(last, STK_ROWS, lo_s + STK_R)
    cnt_p = select(pos_idx, lo_s, hi_s, locr_a, locb_a)
    cnt_n = select(neg_idx, lo_s, hi_s, locr_b, locb_b)
    slists = [(locr_a, locb_a, cnt_p, g_pos), (locr_b, locb_b, cnt_n, g_neg)]

    stream(stk_tab, lo_s, STK_CW,
           jnp.where(last, 37, STK_NCH), slists, STK_G)

    @pl.when(last)
    def _stk_tail():
        pltpu.sync_copy(tail_stk, tail_buf.at[pl.ds(0, 32)])
        consume_tail(STK_TAIL, STK_ROWS - STK_TAIL, slists)


_gather = pl.kernel(
    _gather_body,
    out_type=[
        jax.ShapeDtypeStruct((BATCH + 1, 128), jnp.float32),
        jax.ShapeDtypeStruct((BATCH + 1, 128), jnp.float32),
        jax.ShapeDtypeStruct((BATCH + 1, 128), jnp.float32),
    ],
    mesh=plsc.VectorSubcoreMesh(core_axis_name="c", subcore_axis_name="s"),
    compiler_params=pltpu.CompilerParams(
        needs_layout_passes=False, use_tc_tiling_on_sc=True
    ),
    scratch_types=[
        pltpu.VMEM((BATCH,), jnp.int32),
        pltpu.VMEM((LOC_CAP,), jnp.int32),
        pltpu.VMEM((LOC_CAP,), jnp.int32),
        pltpu.VMEM((LOC_CAP,), jnp.int32),
        pltpu.VMEM((LOC_CAP,), jnp.int32),
        pltpu.VMEM((2, LATENT, INV_CW), jnp.float32),
        pltpu.VMEM((CH_CAP,), jnp.int32),
        pltpu.VMEM((CH_CAP,), jnp.int32),
        pltpu.VMEM((16, 128), jnp.float32),
        pltpu.VMEM((64, LATENT), jnp.float32),
        pltpu.SemaphoreType.DMA,
        pltpu.SemaphoreType.DMA,
    ],
)

BR = 4096


def _dot_body(a_ref, p_ref, n_ref, op_ref, on_ref):
    a = a_ref[...]
    cols = lax.broadcasted_iota(jnp.int32, (BR, 128), 1)
    keep = cols < LATENT
    p = jnp.where(keep, a * p_ref[...], 0.0)
    n = jnp.where(keep, a * n_ref[...], 0.0)
    op_ref[...] = jnp.sum(p, axis=1).reshape(8, 512)
    on_ref[...] = jnp.sum(n, axis=1).reshape(8, 512)


_dot = pl.pallas_call(
    _dot_body,
    grid=(BATCH // BR,),
    in_specs=[
        pl.BlockSpec((BR, 128), lambda i: (i, 0)),
        pl.BlockSpec((BR, 128), lambda i: (i, 0)),
        pl.BlockSpec((BR, 128), lambda i: (i, 0)),
    ],
    out_specs=[
        pl.BlockSpec((8, 512), lambda i: (i, 0)),
        pl.BlockSpec((8, 512), lambda i: (i, 0)),
    ],
    out_shape=[
        jax.ShapeDtypeStruct((BATCH // 512, 512), jnp.float32),
        jax.ShapeDtypeStruct((BATCH // 512, 512), jnp.float32),
    ],
)


@jax.jit
def kernel(investor, stock_positive, stock_negative, embed_investor, embed_stock):
    inv_idx = investor.astype(jnp.int32)
    pos_idx = stock_positive.astype(jnp.int32)
    neg_idx = stock_negative.astype(jnp.int32)
    g_inv, g_pos, g_neg = _gather(
        inv_idx, pos_idx, neg_idx, embed_investor.T, embed_stock.T,
        embed_investor[INV_TAIL:], embed_stock[STK_TAIL:]
    )
    out_p, out_n = _dot(g_inv, g_pos, g_neg)
    return (out_p.reshape(BATCH), out_n.reshape(BATCH))


# speedup vs baseline: 1.8142x; 1.7866x over previous
import jax
import jax.numpy as jnp
from jax import lax
from jax.experimental import pallas as pl
from jax.experimental.pallas import tpu as pltpu
from jax.experimental.pallas import tpu_sc as plsc

BATCH = 16384
LATENT = 64
NC = 2
NS = 16
NW = NC * NS
BPW = BATCH // NW
CHUNK = 128
NCHUNK = BPW // CHUNK


def _body(inv_idx, pos_idx, neg_idx, inv_tab, stk_tab,
          out_pos, out_neg,
          idx_inv, idx_pos, idx_neg,
          inv_rows, pos_rows, neg_rows,
          outp_v, outn_v, sem):
    wid = lax.axis_index("s") * NC + lax.axis_index("c")
    base = wid * BPW

    pltpu.sync_copy(inv_idx.at[wid], idx_inv)
    pltpu.sync_copy(pos_idx.at[wid], idx_pos)
    pltpu.sync_copy(neg_idx.at[wid], idx_neg)

    copies = []
    for j in range(NCHUNK):
        dst = pl.ds(j * CHUNK, CHUNK)
        copies.append(pltpu.async_copy(inv_tab.at[idx_inv.at[j]], inv_rows.at[dst], sem))
        copies.append(pltpu.async_copy(stk_tab.at[idx_pos.at[j]], pos_rows.at[dst], sem))
        copies.append(pltpu.async_copy(stk_tab.at[idx_neg.at[j]], neg_rows.at[dst], sem))
    for c in copies:
        c.wait()

    lanes = lax.broadcasted_iota(jnp.int32, (16,), 0)

    @plsc.parallel_loop(0, BPW // 16)
    def group(g):
        rows16 = g * 16 + lanes
        acc = [jnp.zeros((16,), jnp.float32) for _ in range(8)]
        for d in range(LATENT):
            dcol = jnp.full((16,), d, jnp.int32)
            a = plsc.load_gather(inv_rows, [rows16, dcol])
            p = plsc.load_gather(pos_rows, [rows16, dcol])
            n = plsc.load_gather(neg_rows, [rows16, dcol])
            k = d % 4
            acc[k] = acc[k] + a * p
            acc[4 + k] = acc[4 + k] + a * n
        outp_v[pl.ds(g * 16, 16)] = (acc[0] + acc[1]) + (acc[2] + acc[3])
        outn_v[pl.ds(g * 16, 16)] = (acc[4] + acc[5]) + (acc[6] + acc[7])

    pltpu.sync_copy(outp_v, out_pos.at[pl.ds(base, BPW)])
    pltpu.sync_copy(outn_v, out_neg.at[pl.ds(base, BPW)])


_mf_bpr = pl.kernel(
    _body,
    out_type=[
        jax.ShapeDtypeStruct((BATCH,), jnp.float32),
        jax.ShapeDtypeStruct((BATCH,), jnp.float32),
    ],
    mesh=plsc.VectorSubcoreMesh(core_axis_name="c", subcore_axis_name="s"),
    compiler_params=pltpu.CompilerParams(
        needs_layout_passes=False, use_tc_tiling_on_sc=False
    ),
    scratch_types=[
        pltpu.VMEM((NCHUNK, CHUNK), jnp.int32),
        pltpu.VMEM((NCHUNK, CHUNK), jnp.int32),
        pltpu.VMEM((NCHUNK, CHUNK), jnp.int32),
        pltpu.VMEM((BPW, LATENT), jnp.float32),
        pltpu.VMEM((BPW, LATENT), jnp.float32),
        pltpu.VMEM((BPW, LATENT), jnp.float32),
        pltpu.VMEM((BPW,), jnp.float32),
        pltpu.VMEM((BPW,), jnp.float32),
        pltpu.SemaphoreType.DMA,
    ],
)


@jax.jit
def kernel(investor, stock_positive, stock_negative, embed_investor, embed_stock):
    inv_idx = investor.astype(jnp.int32).reshape(NW, NCHUNK, CHUNK)
    pos_idx = stock_positive.astype(jnp.int32).reshape(NW, NCHUNK, CHUNK)
    neg_idx = stock_negative.astype(jnp.int32).reshape(NW, NCHUNK, CHUNK)
    out_p, out_n = _mf_bpr(inv_idx, pos_idx, neg_idx, embed_investor, embed_stock)
    return (out_p, out_n)
